# baseline (speedup 1.0000x reference)
import functools

import jax
import jax.numpy as jnp
from jax import lax
from jax.experimental import pallas as pl
from jax.experimental.pallas import tpu as pltpu
from jax.experimental.pallas import tpu_sc as plsc

F = 26
B = 4096
V = 100000
D = 64
O = 64
H = 2 * O

VB = 16384
LVB = VB.bit_length() - 1
NVB = -(-V // VB)
RPF = NVB * (VB // 4)

NC = 2
NS = 16
NW = NC * NS
CHUNK = B // NW
L = 16


def _rn16(x):
    return x + 0x7FFF + (lax.shift_right_logical(x, 16) & 1)


def _repack_body(tab_ref, out_ref):
    blk = tab_ref[0]
    bits = lax.bitcast_convert_type(blk, jnp.int32)
    lo = _rn16(bits[: D // 2])
    hi = _rn16(bits[D // 2 :])
    pk = (hi & (-65536)) | lax.shift_right_logical(lo, 16)
    q = VB // 4
    pk128 = jnp.concatenate(
        [pk[:, :q], pk[:, q : 2 * q], pk[:, 2 * q : 3 * q], pk[:, 3 * q :]],
        axis=0,
    )
    out_ref[0] = jnp.transpose(pk128, (1, 0))


def _tc_repack(tab_t, nf):
    return pl.pallas_call(
        _repack_body,
        grid=(nf, NVB),
        in_specs=[pl.BlockSpec((1, D, VB), lambda f, v: (f, 0, v))],
        out_specs=pl.BlockSpec((1, VB // 4, 2 * D), lambda f, v: (f, v, 0)),
        out_shape=jax.ShapeDtypeStruct((nf, RPF, 2 * D), jnp.int32),
    )(tab_t)


def _make_sc_gather_body(nf):
  def _sc_gather_body(idx_hbm, t2_hbm, out_hbm, idx_v, row_v, emb_v,
                        gsem0, gsem1, osem):
      wid = lax.axis_index("s") * NC + lax.axis_index("c")
      base = wid * CHUNK
      gsems = (gsem0, gsem1)

      def load_rows(f, buf):
          pltpu.sync_copy(idx_hbm.at[pl.ds(f * B + base, CHUNK)], idx_v)
          row_off = f * RPF
          for j in range(CHUNK // L):
              sl = pl.ds(j * L, L)
              v = idx_v[sl]
              row_v[buf, sl] = (
                  (lax.shift_right_logical(v, LVB) << (LVB - 2))
                  + (v & (VB // 4 - 1))
                  + row_off
              )

      def gather(f, buf):
          return pltpu.make_async_copy(
              t2_hbm.at[row_v.at[buf]], emb_v.at[buf], gsems[buf])

      def out_copy(f, buf):
          return pltpu.make_async_copy(
              emb_v.at[buf], out_hbm.at[pl.ds(f * B + base, CHUNK)], osem)

      load_rows(0, 0)
      gather(0, 0).start()
      for f in range(nf):
          buf = f % 2
          if f + 1 < nf:
              if f >= 1:
                  out_copy(f - 1, 1 - buf).wait()
              load_rows(f + 1, 1 - buf)
              gather(f + 1, 1 - buf).start()
          gather(f, buf).wait()
          out_copy(f, buf).start()
      out_copy(nf - 2, nf % 2).wait()
      out_copy(nf - 1, 1 - nf % 2).wait()
  return _sc_gather_body


def _sc_gather(idx_flat, t2_flat, nf):
    k = functools.partial(
        pl.kernel,
        mesh=plsc.VectorSubcoreMesh(core_axis_name="c", subcore_axis_name="s"),
        out_type=jax.ShapeDtypeStruct((nf * B, 2 * D), jnp.int32),
        scratch_types=[
            pltpu.VMEM((CHUNK,), jnp.int32),
            pltpu.VMEM((2, CHUNK), jnp.int32),
            pltpu.VMEM((2, CHUNK, 2 * D), jnp.int32),
            pltpu.SemaphoreType.DMA,
            pltpu.SemaphoreType.DMA,
            pltpu.SemaphoreType.DMA,
        ],
        compiler_params=pltpu.CompilerParams(use_tc_tiling_on_sc=True),
    )(_make_sc_gather_body(nf))
    return k(idx_flat, t2_flat)


BB = 4096


def _mlp_body(emb_ref, idx_ref, w1t_ref, b1_ref, w2t_ref, b2_ref, out_ref):
    x = lax.bitcast_convert_type(
        jnp.transpose(lax.bitcast_convert_type(emb_ref[...], jnp.float32),
                      (1, 0)),
        jnp.int32)
    qb = lax.shift_right_logical(idx_ref[0], LVB - 2) & 3
    xa = jnp.where((qb & 1) == 1, x[32:64], x[:32])
    xb = jnp.where((qb & 1) == 1, x[96:128], x[64:96])
    xs = jnp.where(qb >= 2, xb, xa)
    lo_f = lax.bitcast_convert_type(xs << 16, jnp.float32)
    hi_f = lax.bitcast_convert_type(xs & (-65536), jnp.float32)
    embt = jnp.concatenate([lo_f, hi_f], axis=0)
    ht = jnp.dot(w1t_ref[0], embt, preferred_element_type=jnp.float32)
    ht = jnp.maximum(ht + b1_ref[0], 0.0)
    out = jnp.dot(w2t_ref[0], ht, preferred_element_type=jnp.float32)
    out_ref[0] = out + b2_ref[0]


def _tc_mlp(emb2, idx3, W1t, b1r, W2t, b2r, nf):
    return pl.pallas_call(
        _mlp_body,
        grid=(nf, B // BB),
        in_specs=[
            pl.BlockSpec((BB, 2 * D), lambda f, b: (f * (B // BB) + b, 0)),
            pl.BlockSpec((1, 1, BB), lambda f, b: (f, 0, b)),
            pl.BlockSpec((1, H, D), lambda f, b: (f, 0, 0)),
            pl.BlockSpec((1, H, 1), lambda f, b: (f, 0, 0)),
            pl.BlockSpec((1, O, H), lambda f, b: (f, 0, 0)),
            pl.BlockSpec((1, O, 1), lambda f, b: (f, 0, 0)),
        ],
        out_specs=pl.BlockSpec((1, O, BB), lambda f, b: (f, 0, b)),
        out_shape=jax.ShapeDtypeStruct((nf, O, B), jnp.float32),
    )(emb2, idx3, W1t, b1r, W2t, b2r)


def _group(idx, tab_t, W1, b1, W2, b2, lo, hi):
    nf = hi - lo
    t2 = _tc_repack(tab_t[lo:hi], nf)
    emb2 = _sc_gather(idx[lo:hi].reshape(nf * B), t2.reshape(nf * RPF, 2 * D), nf)
    return _tc_mlp(
        emb2,
        idx[lo:hi].reshape(nf, 1, B),
        jnp.transpose(W1[lo:hi], (0, 2, 1)),
        b1[lo:hi].reshape(nf, H, 1),
        jnp.transpose(W2[lo:hi], (0, 2, 1)),
        b2[lo:hi].reshape(nf, O, 1),
        nf,
    )


@jax.jit
def kernel(indices, tables, W1, b1, W2, b2):
    idx = indices.astype(jnp.int32)
    tab_t = jnp.transpose(tables, (0, 2, 1))
    out_a = _group(idx, tab_t, W1, b1, W2, b2, 0, F // 2)
    out_b = _group(idx, tab_t, W1, b1, W2, b2, F // 2, F)
    out_t = jnp.concatenate([out_a, out_b], axis=0)
    return jnp.transpose(out_t, (0, 2, 1))


# speedup vs baseline: 1.8726x; 1.8726x over previous
import functools

import jax
import jax.numpy as jnp
from jax import lax
from jax.experimental import pallas as pl
from jax.experimental.pallas import tpu as pltpu
from jax.experimental.pallas import tpu_sc as plsc

F = 26
B = 4096
V = 100000
D = 64
O = 64
H = 2 * O

VB = 16384
LVB = VB.bit_length() - 1
NVB = -(-V // VB)
RPF = NVB * (VB // 4)

NC = 2
NS = 16
NW = NC * NS
CHUNK = B // NW
L = 16


def _rn16(x):
    return x + 0x7FFF + (lax.shift_right_logical(x, 16) & 1)


def _repack_body(tab_ref, out_ref):
    blk = tab_ref[0]
    bits = lax.bitcast_convert_type(blk, jnp.int32)
    lo = _rn16(bits[: D // 2])
    hi = _rn16(bits[D // 2 :])
    pk = (hi & (-65536)) | lax.shift_right_logical(lo, 16)
    q = VB // 4
    pk128 = jnp.concatenate(
        [pk[:, :q], pk[:, q : 2 * q], pk[:, 2 * q : 3 * q], pk[:, 3 * q :]],
        axis=0,
    )
    out_ref[0] = jnp.transpose(pk128, (1, 0))


def _tc_repack(tab_t):
    return pl.pallas_call(
        _repack_body,
        grid=(F, NVB),
        in_specs=[pl.BlockSpec((1, D, VB), lambda f, v: (f, 0, v))],
        out_specs=pl.BlockSpec((1, VB // 4, 2 * D), lambda f, v: (f, v, 0)),
        out_shape=jax.ShapeDtypeStruct((F, RPF, 2 * D), jnp.int32),
    )(tab_t)


def _sc_gather_body(idx_hbm, t2_hbm, out_hbm, idx_v, row_v, emb_v,
                    gsem0, gsem1, osem):
    wid = lax.axis_index("s") * NC + lax.axis_index("c")
    base = wid * CHUNK
    gsems = (gsem0, gsem1)

    def load_rows(f, buf):
        pltpu.sync_copy(idx_hbm.at[pl.ds(f * B + base, CHUNK)], idx_v)
        row_off = f * RPF
        for j in range(CHUNK // L):
            sl = pl.ds(j * L, L)
            v = idx_v[sl]
            row_v[buf, sl] = (
                (lax.shift_right_logical(v, LVB) << (LVB - 2))
                + (v & (VB // 4 - 1))
                + row_off
            )

    def gather(f, buf):
        return pltpu.make_async_copy(
            t2_hbm.at[row_v.at[buf]], emb_v.at[buf], gsems[buf])

    def out_copy(f, buf):
        return pltpu.make_async_copy(
            emb_v.at[buf], out_hbm.at[pl.ds(f * B + base, CHUNK)], osem)

    load_rows(0, 0)
    gather(0, 0).start()
    for f in range(F):
        buf = f % 2
        if f + 1 < F:
            if f >= 1:
                out_copy(f - 1, 1 - buf).wait()
            load_rows(f + 1, 1 - buf)
            gather(f + 1, 1 - buf).start()
        gather(f, buf).wait()
        out_copy(f, buf).start()
    out_copy(F - 2, 0).wait()
    out_copy(F - 1, 1).wait()


def _sc_gather(idx_flat, t2_flat):
    k = functools.partial(
        pl.kernel,
        mesh=plsc.VectorSubcoreMesh(core_axis_name="c", subcore_axis_name="s"),
        out_type=jax.ShapeDtypeStruct((F * B, 2 * D), jnp.int32),
        scratch_types=[
            pltpu.VMEM((CHUNK,), jnp.int32),
            pltpu.VMEM((2, CHUNK), jnp.int32),
            pltpu.VMEM((2, CHUNK, 2 * D), jnp.int32),
            pltpu.SemaphoreType.DMA,
            pltpu.SemaphoreType.DMA,
            pltpu.SemaphoreType.DMA,
        ],
        compiler_params=pltpu.CompilerParams(use_tc_tiling_on_sc=True),
    )(_sc_gather_body)
    return k(idx_flat, t2_flat)


BB = 4096


def _mlp_body(emb_ref, idx_ref, w1t_ref, b1_ref, w2t_ref, b2_ref, out_ref):
    x = lax.bitcast_convert_type(
        jnp.transpose(lax.bitcast_convert_type(emb_ref[...], jnp.float32),
                      (1, 0)),
        jnp.int32)
    qb = lax.shift_right_logical(idx_ref[0], LVB - 2) & 3
    xa = jnp.where((qb & 1) == 1, x[32:64], x[:32])
    xb = jnp.where((qb & 1) == 1, x[96:128], x[64:96])
    xs = jnp.where(qb >= 2, xb, xa)
    lo_f = lax.bitcast_convert_type(xs << 16, jnp.float32)
    hi_f = lax.bitcast_convert_type(xs & (-65536), jnp.float32)
    embt = jnp.concatenate([lo_f, hi_f], axis=0)
    ht = jnp.dot(w1t_ref[0], embt, preferred_element_type=jnp.float32)
    ht = jnp.maximum(ht + b1_ref[0], 0.0)
    out = jnp.dot(w2t_ref[0], ht, preferred_element_type=jnp.float32)
    out_ref[0] = out + b2_ref[0]


def _tc_mlp(emb2, idx3, W1t, b1r, W2t, b2r):
    return pl.pallas_call(
        _mlp_body,
        grid=(F, B // BB),
        in_specs=[
            pl.BlockSpec((BB, 2 * D), lambda f, b: (f * (B // BB) + b, 0)),
            pl.BlockSpec((1, 1, BB), lambda f, b: (f, 0, b)),
            pl.BlockSpec((1, H, D), lambda f, b: (f, 0, 0)),
            pl.BlockSpec((1, H, 1), lambda f, b: (f, 0, 0)),
            pl.BlockSpec((1, O, H), lambda f, b: (f, 0, 0)),
            pl.BlockSpec((1, O, 1), lambda f, b: (f, 0, 0)),
        ],
        out_specs=pl.BlockSpec((1, O, BB), lambda f, b: (f, 0, b)),
        out_shape=jax.ShapeDtypeStruct((F, O, B), jnp.float32),
    )(emb2, idx3, W1t, b1r, W2t, b2r)


@jax.jit
def kernel(indices, tables, W1, b1, W2, b2):
    idx = indices.astype(jnp.int32)
    tab_t = jnp.transpose(tables, (0, 2, 1))
    t2 = _tc_repack(tab_t)
    emb2 = _sc_gather(idx.reshape(F * B), t2.reshape(F * RPF, 2 * D))
    out_t = _tc_mlp(
        emb2,
        idx.reshape(F, 1, B),
        jnp.transpose(W1, (0, 2, 1)),
        b1.reshape(F, H, 1),
        jnp.transpose(W2, (0, 2, 1)),
        b2.reshape(F, O, 1),
    )
    return jnp.transpose(out_t, (0, 2, 1))


# speedup vs baseline: 1.8964x; 1.0127x over previous
import functools

import jax
import jax.numpy as jnp
from jax import lax
from jax.experimental import pallas as pl
from jax.experimental.pallas import tpu as pltpu
from jax.experimental.pallas import tpu_sc as plsc

F = 26
B = 4096
V = 100000
D = 64
O = 64
H = 2 * O

VB = 16384
LVB = VB.bit_length() - 1
NVB = -(-V // VB)
RPF = NVB * (VB // 4)

NC = 2
NS = 16
NW = NC * NS
CHUNK = B // NW
L = 16


def _rn16(x):
    return x + 0x7FFF + (lax.shift_right_logical(x, 16) & 1)


def _repack_body(tab_ref, out_ref):
    blk = tab_ref[0]
    bits = lax.bitcast_convert_type(blk, jnp.int32)
    lo = _rn16(bits[: D // 2])
    hi = _rn16(bits[D // 2 :])
    pk = (hi & (-65536)) | lax.shift_right_logical(lo, 16)
    q = VB // 4
    pk128 = jnp.concatenate(
        [pk[:, :q], pk[:, q : 2 * q], pk[:, 2 * q : 3 * q], pk[:, 3 * q :]],
        axis=0,
    )
    out_ref[0] = jnp.transpose(pk128, (1, 0))


def _tc_repack(tab_t):
    return pl.pallas_call(
        _repack_body,
        grid=(F, NVB),
        in_specs=[pl.BlockSpec((1, D, VB), lambda f, v: (f, 0, v))],
        out_specs=pl.BlockSpec((1, VB // 4, 2 * D), lambda f, v: (f, v, 0)),
        out_shape=jax.ShapeDtypeStruct((F, RPF, 2 * D), jnp.int32),
    )(tab_t)


def _sc_gather_body(idx_hbm, t2_hbm, out_hbm, idx_v, row_v, emb_v,
                    gsem0, gsem1, gsem2, osem0, osem1, osem2):
    wid = lax.axis_index("s") * NC + lax.axis_index("c")
    base = wid * CHUNK
    gsems = (gsem0, gsem1, gsem2)
    osems = (osem0, osem1, osem2)

    def load_rows(f, buf):
        pltpu.sync_copy(idx_hbm.at[pl.ds(f * B + base, CHUNK)], idx_v)
        row_off = f * RPF
        for j in range(CHUNK // L):
            sl = pl.ds(j * L, L)
            v = idx_v[sl]
            row_v[buf, sl] = (
                (lax.shift_right_logical(v, LVB) << (LVB - 2))
                + (v & (VB // 4 - 1))
                + row_off
            )

    def gather(f, buf):
        return pltpu.make_async_copy(
            t2_hbm.at[row_v.at[buf]], emb_v.at[buf], gsems[buf])

    def out_copy(f, buf):
        return pltpu.make_async_copy(
            emb_v.at[buf], out_hbm.at[pl.ds(f * B + base, CHUNK)], osems[buf])

    load_rows(0, 0)
    gather(0, 0).start()
    load_rows(1, 1)
    gather(1, 1).start()
    for f in range(F):
        buf = f % 3
        if f + 2 < F:
            nb = (f + 2) % 3
            if f >= 1:
                out_copy(f - 1, nb).wait()
            load_rows(f + 2, nb)
            gather(f + 2, nb).start()
        gather(f, buf).wait()
        out_copy(f, buf).start()
    out_copy(F - 3, (F - 3) % 3).wait()
    out_copy(F - 2, (F - 2) % 3).wait()
    out_copy(F - 1, (F - 1) % 3).wait()


def _sc_gather(idx_flat, t2_flat):
    k = functools.partial(
        pl.kernel,
        mesh=plsc.VectorSubcoreMesh(core_axis_name="c", subcore_axis_name="s"),
        out_type=jax.ShapeDtypeStruct((F * B, 2 * D), jnp.int32),
        scratch_types=[
            pltpu.VMEM((CHUNK,), jnp.int32),
            pltpu.VMEM((3, CHUNK), jnp.int32),
            pltpu.VMEM((3, CHUNK, 2 * D), jnp.int32),
            pltpu.SemaphoreType.DMA,
            pltpu.SemaphoreType.DMA,
            pltpu.SemaphoreType.DMA,
            pltpu.SemaphoreType.DMA,
            pltpu.SemaphoreType.DMA,
            pltpu.SemaphoreType.DMA,
        ],
        compiler_params=pltpu.CompilerParams(use_tc_tiling_on_sc=True),
    )(_sc_gather_body)
    return k(idx_flat, t2_flat)


BB = 4096


def _mlp_body(emb_ref, idx_ref, w1t_ref, b1_ref, w2t_ref, b2_ref, out_ref):
    x = lax.bitcast_convert_type(
        jnp.transpose(lax.bitcast_convert_type(emb_ref[...], jnp.float32),
                      (1, 0)),
        jnp.int32)
    qb = lax.shift_right_logical(idx_ref[0], LVB - 2) & 3
    xa = jnp.where((qb & 1) == 1, x[32:64], x[:32])
    xb = jnp.where((qb & 1) == 1, x[96:128], x[64:96])
    xs = jnp.where(qb >= 2, xb, xa)
    lo_f = lax.bitcast_convert_type(xs << 16, jnp.float32)
    hi_f = lax.bitcast_convert_type(xs & (-65536), jnp.float32)
    embt = jnp.concatenate([lo_f, hi_f], axis=0)
    ht = jnp.dot(w1t_ref[0], embt, preferred_element_type=jnp.float32)
    ht = jnp.maximum(ht + b1_ref[0], 0.0)
    out = jnp.dot(w2t_ref[0], ht, preferred_element_type=jnp.float32)
    out_ref[0] = out + b2_ref[0]


def _tc_mlp(emb2, idx3, W1t, b1r, W2t, b2r):
    return pl.pallas_call(
        _mlp_body,
        grid=(F, B // BB),
        in_specs=[
            pl.BlockSpec((BB, 2 * D), lambda f, b: (f * (B // BB) + b, 0)),
            pl.BlockSpec((1, 1, BB), lambda f, b: (f, 0, b)),
            pl.BlockSpec((1, H, D), lambda f, b: (f, 0, 0)),
            pl.BlockSpec((1, H, 1), lambda f, b: (f, 0, 0)),
            pl.BlockSpec((1, O, H), lambda f, b: (f, 0, 0)),
            pl.BlockSpec((1, O, 1), lambda f, b: (f, 0, 0)),
        ],
        out_specs=pl.BlockSpec((1, O, BB), lambda f, b: (f, 0, b)),
        out_shape=jax.ShapeDtypeStruct((F, O, B), jnp.float32),
    )(emb2, idx3, W1t, b1r, W2t, b2r)


@jax.jit
def kernel(indices, tables, W1, b1, W2, b2):
    idx = indices.astype(jnp.int32)
    tab_t = jnp.transpose(tables, (0, 2, 1))
    t2 = _tc_repack(tab_t)
    emb2 = _sc_gather(idx.reshape(F * B), t2.reshape(F * RPF, 2 * D))
    out_t = _tc_mlp(
        emb2,
        idx.reshape(F, 1, B),
        jnp.transpose(W1, (0, 2, 1)),
        b1.reshape(F, H, 1),
        jnp.transpose(W2, (0, 2, 1)),
        b2.reshape(F, O, 1),
    )
    return jnp.transpose(out_t, (0, 2, 1))
